# baseline (speedup 1.0000x reference)
import functools

import jax
import jax.numpy as jnp
from jax import lax
from jax.experimental import pallas as pl
from jax.experimental.pallas import tpu as pltpu
from jax.experimental.pallas import tpu_sc as plsc

B, T, D = 4096, 200, 32
NC, NS = 2, 16
NW = NC * NS
JB = B // NW
TT = 4
NBLK = T // TT
ROWLEN = 4 * NW * 1024

_mesh = plsc.VectorSubcoreMesh(core_axis_name="c", subcore_axis_name="s")


@functools.partial(
    pl.kernel,
    mesh=_mesh,
    out_type=[
        jax.ShapeDtypeStruct((T, ROWLEN), jnp.float32),
        jax.ShapeDtypeStruct((T, ROWLEN), jnp.float32),
    ],
    scratch_types=[
        [pltpu.VMEM((TT, 1, JB), jnp.int32) for _ in range(2)],
        [pltpu.VMEM((TT, 1, JB), jnp.int32) for _ in range(2)],
        [pltpu.VMEM((JB, D), jnp.float32) for _ in range(2)],
        [pltpu.VMEM((JB, D), jnp.float32) for _ in range(2)],
        [pltpu.VMEM((4, TT, 1024), jnp.float32) for _ in range(2)],
        [pltpu.VMEM((4, TT, 1024), jnp.float32) for _ in range(2)],
        [pltpu.SemaphoreType.DMA for _ in range(2)],
        [pltpu.SemaphoreType.DMA for _ in range(2)],
        [pltpu.SemaphoreType.DMA for _ in range(2)],
        [pltpu.SemaphoreType.DMA for _ in range(2)],
    ],
    compiler_params=pltpu.CompilerParams(
        use_tc_tiling_on_sc=False, needs_layout_passes=False),
)
def _gather_pair(item_idx, cat_idx, w_item, w_cat, out_i, out_c,
                 idx_i, idx_c, g_i, g_c, o_i, o_c,
                 gsem_i, gsem_c, wsem_i, wsem_c):
    w = lax.axis_index("s") * NC + lax.axis_index("c")

    iota16 = lax.iota(jnp.int32, 16)
    rows16 = [iota16 + 16 * k for k in range(8)]

    def stage_idx(blk, p):
        pltpu.sync_copy(item_idx.at[pl.ds(blk * TT, TT), pl.ds(w, 1)], idx_i[p])
        pltpu.sync_copy(cat_idx.at[pl.ds(blk * TT, TT), pl.ds(w, 1)], idx_c[p])

    def fire(ip, tt, gp):
        pltpu.async_copy(w_item.at[idx_i[ip].at[tt, 0]], g_i[gp], gsem_i[gp])
        pltpu.async_copy(w_cat.at[idx_c[ip].at[tt, 0]], g_c[gp], gsem_c[gp])

    def transpose_t(gp, ob, tt):
        for q in range(4):
            def body(r, _):
                col = r * 128
                for g_buf, o_buf in ((g_i, o_i), (g_c, o_c)):
                    d16 = jnp.full((16,), q * 8 + r, jnp.int32)
                    for k in range(8):
                        v = plsc.load_gather(g_buf[gp], [rows16[k], d16])
                        o_buf[ob][q, tt, pl.ds(col + 16 * k, 16)] = v
                return ()
            lax.fori_loop(0, 8, body, ())

    def drain_gathers(gp):
        pltpu.make_async_copy(out_i.at[pl.ds(0, JB), pl.ds(0, D)],
                              g_i[gp], gsem_i[gp]).wait()
        pltpu.make_async_copy(out_c.at[pl.ds(0, JB), pl.ds(0, D)],
                              g_c[gp], gsem_c[gp]).wait()

    def drain_writes(ob):
        for q in range(4):
            pltpu.make_async_copy(o_i[ob].at[q],
                                  out_i.at[pl.ds(0, TT), pl.ds(0, 1024)],
                                  wsem_i[ob]).wait()
            pltpu.make_async_copy(o_c[ob].at[q],
                                  out_c.at[pl.ds(0, TT), pl.ds(0, 1024)],
                                  wsem_c[ob]).wait()

    def fire_writes(blk, ob):
        t0 = blk * TT
        for q in range(4):
            off = (q * NW + w) * 1024
            pltpu.async_copy(o_i[ob].at[q],
                             out_i.at[pl.ds(t0, TT), pl.ds(off, 1024)], wsem_i[ob])
            pltpu.async_copy(o_c[ob].at[q],
                             out_c.at[pl.ds(t0, TT), pl.ds(off, 1024)], wsem_c[ob])

    stage_idx(0, 0)
    fire(0, 0, 0)

    def two_blocks(i, _):
        for ob in range(2):
            blk = 2 * i + ob

            @pl.when(blk + 1 < NBLK)
            def _():
                stage_idx(blk + 1, ob ^ 1)

            @pl.when(blk >= 2)
            def _():
                drain_writes(ob)

            for tt in range(TT):
                gp = tt % 2

                @pl.when(blk * TT + tt + 1 < T)
                def _(tt=tt, gp=gp, ob=ob):
                    if tt + 1 < TT:
                        fire(ob, tt + 1, gp ^ 1)
                    else:
                        fire(ob ^ 1, 0, gp ^ 1)

                drain_gathers(gp)
                transpose_t(gp, ob, tt)

            fire_writes(blk, ob)
        return ()

    lax.fori_loop(0, NBLK // 2, two_blocks, ())
    drain_writes(0)
    drain_writes(1)


def kernel(item_id, cat_id, price, discount, W_item, W_cat):
    item_idx = item_id.T.reshape(T, NW, JB).astype(jnp.int32)
    cat_idx = cat_id.T.reshape(T, NW, JB).astype(jnp.int32)
    li, lc = _gather_pair(item_idx, cat_idx, W_item, W_cat)

    def unpack(l):
        return (l.reshape(T, 4, NW, 8, JB)
                 .transpose(2, 4, 0, 1, 3)
                 .reshape(B, T, D))

    return (unpack(li), unpack(lc), price[..., None], discount[..., None])
